# baseline (speedup 1.0000x reference)
import functools

import jax
import jax.numpy as jnp
from jax import lax
from jax.experimental import pallas as pl
from jax.experimental.pallas import tpu as pltpu
from jax.experimental.pallas import tpu_sc as plsc

_T = 4.0
_NB = 64
_ROWS = 8
_NPR = 96 * 32 * 32
_SPLIT = 4
_CHUNK = _NPR // _SPLIT
_ITERS = _CHUNK // 16
_NC = 2
_NS = 16


def _sc_hist_kernel(fs_hbm, ft_hbm, out_hbm, s_v, t_v, hist_v, res_v, sem):
    wid = lax.axis_index("s") * _NC + lax.axis_index("c")
    row = wid // _SPLIT
    q = wid % _SPLIT
    off = row * _NPR + q * _CHUNK
    cs = pltpu.async_copy(fs_hbm.at[pl.ds(off, _CHUNK)], s_v, sem)
    ct = pltpu.async_copy(ft_hbm.at[pl.ds(off, _CHUNK)], t_v, sem)

    zeros = jnp.zeros((16,), jnp.float32)
    for i in range((2 * 16 * _NB) // 16):
        hist_v[pl.ds(i * 16, 16)] = zeros
    cs.wait()
    ct.wait()

    lane = lax.broadcasted_iota(jnp.int32, (16,), 0)
    base_s = lane * _NB
    base_t = base_s + 16 * _NB
    ones = jnp.ones((16,), jnp.float32)
    unroll = 8

    def body(i, carry):
        base = i * (16 * unroll)
        for u in range(unroll):
            vs = s_v[pl.ds(base + u * 16, 16)].astype(jnp.int32)
            plsc.addupdate_scatter(hist_v, [base_s + vs], ones)
        for u in range(unroll):
            vt = t_v[pl.ds(base + u * 16, 16)].astype(jnp.int32)
            plsc.addupdate_scatter(hist_v, [base_t + vt], ones)
        return carry

    lax.fori_loop(0, _ITERS // unroll, body, 0)

    for t in range(2):
        for j in range(4):
            acc = hist_v[pl.ds(t * 16 * _NB + j * 16, 16)]
            for l in range(1, 16):
                acc = acc + hist_v[pl.ds(t * 16 * _NB + l * _NB + j * 16, 16)]
            res_v[pl.ds(t * _NB + j * 16, 16)] = acc

    base = (q * 16 + row) * _NB
    pltpu.sync_copy(res_v.at[pl.ds(0, _NB)], out_hbm.at[pl.ds(base, _NB)])
    pltpu.sync_copy(res_v.at[pl.ds(_NB, _NB)],
                    out_hbm.at[pl.ds(base + 8 * _NB, _NB)])


@functools.cache
def _hist_call():
    return pl.kernel(
        _sc_hist_kernel,
        out_type=jax.ShapeDtypeStruct((4 * 16 * _NB,), jnp.float32),
        mesh=plsc.VectorSubcoreMesh(core_axis_name="c", subcore_axis_name="s"),
        scratch_types=[
            pltpu.VMEM((_CHUNK,), jnp.float32),
            pltpu.VMEM((_CHUNK,), jnp.float32),
            pltpu.VMEM((2 * 16 * _NB,), jnp.float32),
            pltpu.VMEM((2 * _NB,), jnp.float32),
            pltpu.SemaphoreType.DMA,
        ],
        compiler_params=pltpu.CompilerParams(needs_layout_passes=False),
    )


def _compact_counts(h):
    ones8 = jnp.ones((1, _ROWS), jnp.float32)
    tot = jnp.dot(ones8, h, preferred_element_type=jnp.float32,
                  precision=lax.Precision.HIGHEST)
    pres = (tot > 0.5).astype(jnp.float32)
    iota_w = lax.broadcasted_iota(jnp.int32, (_NB, _NB), 0)
    iota_v = lax.broadcasted_iota(jnp.int32, (_NB, _NB), 1)
    le = (iota_w <= iota_v).astype(jnp.float32)
    cum = jnp.dot(pres, le, preferred_element_type=jnp.float32,
                  precision=lax.Precision.HIGHEST)
    iota_i = lax.broadcasted_iota(jnp.int32, (_NB, _NB), 0).astype(jnp.float32)
    q = pres * jnp.where(jnp.abs(cum - (iota_i + 1.0)) < 0.5, 1.0, 0.0)
    comp = lax.dot_general(h, q, (((1,), (1,)), ((), ())),
                           preferred_element_type=jnp.float32,
                           precision=lax.Precision.HIGHEST)
    comp = jnp.floor(comp + 0.5)
    nb = jnp.sum(pres)
    one_bin = nb == 1.0
    iota_row = lax.broadcasted_iota(jnp.int32, (1, _NB), 1).astype(jnp.float32)
    valid = jnp.where(one_bin,
                      jnp.where(iota_row < 2.0, 1.0, 0.0),
                      jnp.where(iota_row < nb, 1.0, 0.0))
    one = jnp.where(one_bin, jnp.ones((1, _NB), jnp.float32),
                    jnp.zeros((1, _NB), jnp.float32))
    return comp, valid, one


def _compact_kernel(p_ref, counts_ref, meta_ref):
    hist = p_ref[0] + p_ref[1] + p_ref[2] + p_ref[3]
    comp_s, valid_s, one_s = _compact_counts(hist[0:_ROWS])
    comp_t, valid_t, one_t = _compact_counts(hist[_ROWS:2 * _ROWS])
    counts_ref[0:_ROWS] = comp_s
    counts_ref[_ROWS:2 * _ROWS] = comp_t
    meta_ref[0:1] = valid_s
    meta_ref[1:2] = valid_t
    meta_ref[2:3] = one_s
    meta_ref[3:4] = one_t


def kernel(feat_s, feat_t):
    fs = jnp.transpose(feat_s, (0, 2, 3, 1)).reshape(-1)
    ft = jnp.transpose(feat_t, (0, 2, 3, 1)).reshape(-1)
    partials = _hist_call()(fs, ft).reshape(4, 16, _NB)
    counts, meta = pl.pallas_call(
        _compact_kernel,
        out_shape=[
            jax.ShapeDtypeStruct((2 * _ROWS, _NB), jnp.float32),
            jax.ShapeDtypeStruct((4, _NB), jnp.float32),
        ],
    )(partials)
    comp_s = counts[0:_ROWS]
    comp_t = counts[_ROWS:2 * _ROWS]
    valid_s = meta[0] > 0.5
    valid_t = meta[1] > 0.5
    one_s = meta[2, 0] > 0.5
    one_t = meta[3, 0] > 0.5
    logit_s = jnp.where(one_s, jnp.zeros_like(comp_s),
                        jnp.log(comp_s + 1e-8))
    logit_t = jnp.where(one_t, jnp.zeros_like(comp_t),
                        jnp.log(comp_t + 1e-8))
    p_s = jax.nn.log_softmax(jnp.where(valid_s, logit_s / _T, -jnp.inf),
                             axis=1)
    p_t = jax.nn.softmax(jnp.where(valid_t, logit_t / _T, -jnp.inf), axis=1)
    valid = valid_s & valid_t
    kl_terms = jnp.where(valid, p_t * (jnp.log(p_t) - p_s), 0.0)
    return jnp.sum(kl_terms) * (_T ** 2) / _ROWS


# speedup vs baseline: 1.0109x; 1.0109x over previous
import functools

import jax
import jax.numpy as jnp
from jax import lax
from jax.experimental import pallas as pl
from jax.experimental.pallas import tpu as pltpu
from jax.experimental.pallas import tpu_sc as plsc

_T = 4.0
_NB = 64
_ROWS = 8
_NPR = 96 * 32 * 32
_SPLIT = 4
_CHUNK = _NPR // _SPLIT
_ITERS = _CHUNK // 16
_NC = 2
_NS = 16


def _sc_hist_kernel(fs_hbm, ft_hbm, out_hbm, s_v, t_v, hist_v, res_v, sem):
    wid = lax.axis_index("s") * _NC + lax.axis_index("c")
    row = wid // _SPLIT
    q = wid % _SPLIT
    off = row * _NPR + q * _CHUNK
    cs = pltpu.async_copy(fs_hbm.at[pl.ds(off, _CHUNK)], s_v, sem)
    ct = pltpu.async_copy(ft_hbm.at[pl.ds(off, _CHUNK)], t_v, sem)

    zeros = jnp.zeros((16,), jnp.float32)
    for i in range((4 * 16 * _NB) // 16):
        hist_v[pl.ds(i * 16, 16)] = zeros
    cs.wait()
    ct.wait()

    lane = lax.broadcasted_iota(jnp.int32, (16,), 0)
    lane64 = lane * _NB
    region = 16 * _NB
    ones = jnp.ones((16,), jnp.float32)
    unroll = 4

    def body(i, carry):
        base = i * (16 * unroll)
        for u in range(unroll):
            off_u = base + u * 16
            par = u % 2
            vs = s_v[pl.ds(off_u, 16)].astype(jnp.int32)
            plsc.addupdate_scatter(hist_v, [lane64 + (par * 2 * region) + vs],
                                   ones)
            vt = t_v[pl.ds(off_u, 16)].astype(jnp.int32)
            plsc.addupdate_scatter(
                hist_v, [lane64 + (par * 2 * region + region) + vt], ones)
        return carry

    lax.fori_loop(0, _ITERS // unroll, body, 0)

    for t in range(2):
        for j in range(4):
            acc = hist_v[pl.ds(t * region + j * 16, 16)]
            for p in range(2):
                for l in range(16):
                    if p == 0 and l == 0:
                        continue
                    acc = acc + hist_v[pl.ds(p * 2 * region + t * region
                                             + l * _NB + j * 16, 16)]
            res_v[pl.ds(t * _NB + j * 16, 16)] = acc

    pltpu.sync_copy(res_v, out_hbm.at[pl.ds((q * 8 + row) * 128, 128)])


@functools.cache
def _hist_call():
    return pl.kernel(
        _sc_hist_kernel,
        out_type=jax.ShapeDtypeStruct((4 * 16 * _NB,), jnp.float32),
        mesh=plsc.VectorSubcoreMesh(core_axis_name="c", subcore_axis_name="s"),
        scratch_types=[
            pltpu.VMEM((_CHUNK,), jnp.float32),
            pltpu.VMEM((_CHUNK,), jnp.float32),
            pltpu.VMEM((4 * 16 * _NB,), jnp.float32),
            pltpu.VMEM((2 * _NB,), jnp.float32),
            pltpu.SemaphoreType.DMA,
        ],
        compiler_params=pltpu.CompilerParams(needs_layout_passes=False),
    )


def _compact_counts(h):
    ones8 = jnp.ones((1, _ROWS), jnp.float32)
    tot = jnp.dot(ones8, h, preferred_element_type=jnp.float32,
                  precision=lax.Precision.HIGHEST)
    pres = (tot > 0.5).astype(jnp.float32)
    iota_w = lax.broadcasted_iota(jnp.int32, (_NB, _NB), 0)
    iota_v = lax.broadcasted_iota(jnp.int32, (_NB, _NB), 1)
    le = (iota_w <= iota_v).astype(jnp.float32)
    cum = jnp.dot(pres, le, preferred_element_type=jnp.float32,
                  precision=lax.Precision.HIGHEST)
    iota_i = lax.broadcasted_iota(jnp.int32, (_NB, _NB), 0).astype(jnp.float32)
    q = pres * jnp.where(jnp.abs(cum - (iota_i + 1.0)) < 0.5, 1.0, 0.0)
    comp = lax.dot_general(h, q, (((1,), (1,)), ((), ())),
                           preferred_element_type=jnp.float32,
                           precision=lax.Precision.HIGHEST)
    comp = jnp.floor(comp + 0.5)
    nb = jnp.sum(pres)
    one_bin = nb == 1.0
    iota_row = lax.broadcasted_iota(jnp.int32, (1, _NB), 1).astype(jnp.float32)
    valid = jnp.where(one_bin,
                      jnp.where(iota_row < 2.0, 1.0, 0.0),
                      jnp.where(iota_row < nb, 1.0, 0.0))
    one = jnp.where(one_bin, jnp.ones((1, _NB), jnp.float32),
                    jnp.zeros((1, _NB), jnp.float32))
    return comp, valid, one


def _compact_kernel(p_ref, counts_ref, meta_ref):
    p2 = (p_ref[0:8] + p_ref[8:16] + p_ref[16:24] + p_ref[24:32])
    comp_s, valid_s, one_s = _compact_counts(p2[:, 0:_NB])
    comp_t, valid_t, one_t = _compact_counts(p2[:, _NB:2 * _NB])
    counts_ref[0:_ROWS] = comp_s
    counts_ref[_ROWS:2 * _ROWS] = comp_t
    meta_ref[0:1] = valid_s
    meta_ref[1:2] = valid_t
    meta_ref[2:3] = one_s
    meta_ref[3:4] = one_t


def kernel(feat_s, feat_t):
    fs = jnp.transpose(feat_s, (0, 2, 3, 1)).reshape(-1)
    ft = jnp.transpose(feat_t, (0, 2, 3, 1)).reshape(-1)
    partials = _hist_call()(fs, ft).reshape(32, 128)
    counts, meta = pl.pallas_call(
        _compact_kernel,
        out_shape=[
            jax.ShapeDtypeStruct((2 * _ROWS, _NB), jnp.float32),
            jax.ShapeDtypeStruct((4, _NB), jnp.float32),
        ],
    )(partials)
    comp_s = counts[0:_ROWS]
    comp_t = counts[_ROWS:2 * _ROWS]
    valid_s = meta[0] > 0.5
    valid_t = meta[1] > 0.5
    one_s = meta[2, 0] > 0.5
    one_t = meta[3, 0] > 0.5
    logit_s = jnp.where(one_s, jnp.zeros_like(comp_s),
                        jnp.log(comp_s + 1e-8))
    logit_t = jnp.where(one_t, jnp.zeros_like(comp_t),
                        jnp.log(comp_t + 1e-8))
    p_s = jax.nn.log_softmax(jnp.where(valid_s, logit_s / _T, -jnp.inf),
                             axis=1)
    p_t = jax.nn.softmax(jnp.where(valid_t, logit_t / _T, -jnp.inf), axis=1)
    valid = valid_s & valid_t
    kl_terms = jnp.where(valid, p_t * (jnp.log(p_t) - p_s), 0.0)
    return jnp.sum(kl_terms) * (_T ** 2) / _ROWS


# speedup vs baseline: 1.4226x; 1.4072x over previous
import functools

import jax
import jax.numpy as jnp
from jax import lax
from jax.experimental import pallas as pl
from jax.experimental.pallas import tpu as pltpu
from jax.experimental.pallas import tpu_sc as plsc

_T = 4.0
_NB = 64
_ROWS = 8
_NPR = 96 * 32 * 32
_SPLIT = 4
_CHUNK = _NPR // _SPLIT
_ITERS = _CHUNK // 16
_NC = 2
_NS = 16


def _sc_hist_kernel(fs_hbm, ft_hbm, out_hbm, s_v, t_v, hist_v, res_v, sem):
    wid = lax.axis_index("s") * _NC + lax.axis_index("c")
    row = wid // _SPLIT
    q = wid % _SPLIT
    off = row * _NPR + q * _CHUNK
    cs = pltpu.async_copy(fs_hbm.at[pl.ds(off, _CHUNK)], s_v, sem)
    ct = pltpu.async_copy(ft_hbm.at[pl.ds(off, _CHUNK)], t_v, sem)

    zeros = jnp.zeros((16,), jnp.float32)
    for i in range((4 * 16 * _NB) // 16):
        hist_v[pl.ds(i * 16, 16)] = zeros
    cs.wait()
    ct.wait()

    lane = lax.broadcasted_iota(jnp.int32, (16,), 0)
    lane64 = lane * _NB
    region = 16 * _NB
    ones = jnp.ones((16,), jnp.float32)
    unroll = 4

    def body(i, carry):
        base = i * (16 * unroll)
        vals = []
        for u in range(unroll):
            vals.append(s_v[pl.ds(base + u * 16, 16)])
            vals.append(t_v[pl.ds(base + u * 16, 16)])
        idxs = []
        for k, v in enumerate(vals):
            par = (k // 2) % 2
            tens = k % 2
            idxs.append(lane64 + (par * 2 * region + tens * region)
                        + v.astype(jnp.int32))
        for idx in idxs:
            plsc.addupdate_scatter(hist_v, [idx], ones)
        return carry

    lax.fori_loop(0, _ITERS // unroll, body, 0)

    for t in range(2):
        for j in range(4):
            acc = hist_v[pl.ds(t * region + j * 16, 16)]
            for p in range(2):
                for l in range(16):
                    if p == 0 and l == 0:
                        continue
                    acc = acc + hist_v[pl.ds(p * 2 * region + t * region
                                             + l * _NB + j * 16, 16)]
            res_v[pl.ds(t * _NB + j * 16, 16)] = acc

    pltpu.sync_copy(res_v, out_hbm.at[pl.ds((q * 8 + row) * 128, 128)])


@functools.cache
def _hist_call():
    return pl.kernel(
        _sc_hist_kernel,
        out_type=jax.ShapeDtypeStruct((4 * 16 * _NB,), jnp.float32),
        mesh=plsc.VectorSubcoreMesh(core_axis_name="c", subcore_axis_name="s"),
        scratch_types=[
            pltpu.VMEM((_CHUNK,), jnp.float32),
            pltpu.VMEM((_CHUNK,), jnp.float32),
            pltpu.VMEM((4 * 16 * _NB,), jnp.float32),
            pltpu.VMEM((2 * _NB,), jnp.float32),
            pltpu.SemaphoreType.DMA,
        ],
        compiler_params=pltpu.CompilerParams(needs_layout_passes=False),
    )


def _compact_counts(h):
    ones8 = jnp.ones((1, _ROWS), jnp.float32)
    tot = jnp.dot(ones8, h, preferred_element_type=jnp.float32,
                  precision=lax.Precision.HIGHEST)
    pres = (tot > 0.5).astype(jnp.float32)
    iota_w = lax.broadcasted_iota(jnp.int32, (_NB, _NB), 0)
    iota_v = lax.broadcasted_iota(jnp.int32, (_NB, _NB), 1)
    le = (iota_w <= iota_v).astype(jnp.float32)
    cum = jnp.dot(pres, le, preferred_element_type=jnp.float32,
                  precision=lax.Precision.HIGHEST)
    iota_i = lax.broadcasted_iota(jnp.int32, (_NB, _NB), 0).astype(jnp.float32)
    q = pres * jnp.where(jnp.abs(cum - (iota_i + 1.0)) < 0.5, 1.0, 0.0)
    comp = lax.dot_general(h, q, (((1,), (1,)), ((), ())),
                           preferred_element_type=jnp.float32,
                           precision=lax.Precision.HIGHEST)
    comp = jnp.floor(comp + 0.5)
    nb = jnp.sum(pres)
    one_bin = nb == 1.0
    iota_row = lax.broadcasted_iota(jnp.int32, (1, _NB), 1).astype(jnp.float32)
    valid = jnp.where(one_bin,
                      jnp.where(iota_row < 2.0, 1.0, 0.0),
                      jnp.where(iota_row < nb, 1.0, 0.0))
    one = jnp.where(one_bin, jnp.ones((1, _NB), jnp.float32),
                    jnp.zeros((1, _NB), jnp.float32))
    return comp, valid, one


def _compact_kernel(p_ref, counts_ref, meta_ref):
    p2 = (p_ref[0:8] + p_ref[8:16] + p_ref[16:24] + p_ref[24:32])
    comp_s, valid_s, one_s = _compact_counts(p2[:, 0:_NB])
    comp_t, valid_t, one_t = _compact_counts(p2[:, _NB:2 * _NB])
    counts_ref[0:_ROWS] = comp_s
    counts_ref[_ROWS:2 * _ROWS] = comp_t
    meta_ref[0:1] = valid_s
    meta_ref[1:2] = valid_t
    meta_ref[2:3] = one_s
    meta_ref[3:4] = one_t


def kernel(feat_s, feat_t):
    fs = jnp.transpose(feat_s, (0, 2, 3, 1)).reshape(-1)
    ft = jnp.transpose(feat_t, (0, 2, 3, 1)).reshape(-1)
    partials = _hist_call()(fs, ft).reshape(32, 128)
    counts, meta = pl.pallas_call(
        _compact_kernel,
        out_shape=[
            jax.ShapeDtypeStruct((2 * _ROWS, _NB), jnp.float32),
            jax.ShapeDtypeStruct((4, _NB), jnp.float32),
        ],
    )(partials)
    comp_s = counts[0:_ROWS]
    comp_t = counts[_ROWS:2 * _ROWS]
    valid_s = meta[0] > 0.5
    valid_t = meta[1] > 0.5
    one_s = meta[2, 0] > 0.5
    one_t = meta[3, 0] > 0.5
    logit_s = jnp.where(one_s, jnp.zeros_like(comp_s),
                        jnp.log(comp_s + 1e-8))
    logit_t = jnp.where(one_t, jnp.zeros_like(comp_t),
                        jnp.log(comp_t + 1e-8))
    p_s = jax.nn.log_softmax(jnp.where(valid_s, logit_s / _T, -jnp.inf),
                             axis=1)
    p_t = jax.nn.softmax(jnp.where(valid_t, logit_t / _T, -jnp.inf), axis=1)
    valid = valid_s & valid_t
    kl_terms = jnp.where(valid, p_t * (jnp.log(p_t) - p_s), 0.0)
    return jnp.sum(kl_terms) * (_T ** 2) / _ROWS


# speedup vs baseline: 1.6726x; 1.1758x over previous
import functools

import jax
import jax.numpy as jnp
from jax import lax
from jax.experimental import pallas as pl
from jax.experimental.pallas import tpu as pltpu
from jax.experimental.pallas import tpu_sc as plsc

_T = 4.0
_NB = 64
_ROWS = 8
_NPR = 96 * 32 * 32
_SPLIT = 4
_CHUNK = _NPR // _SPLIT
_ITERS = _CHUNK // 16
_NC = 2
_NS = 16


_RPW = 256


def _sc_hist_kernel(fs_hbm, ft_hbm, out_hbm, s_v, t_v, hist_v, res_v, sem):
    wid = lax.axis_index("s") * _NC + lax.axis_index("c")
    row = wid // _SPLIT
    q = wid % _SPLIT
    r0 = wid * _RPW
    cs = pltpu.async_copy(fs_hbm.at[pl.ds(r0, _RPW), :], s_v, sem)
    ct = pltpu.async_copy(ft_hbm.at[pl.ds(r0, _RPW), :], t_v, sem)

    zeros = jnp.zeros((16,), jnp.float32)
    for i in range((4 * 16 * _NB) // 16):
        hist_v[pl.ds(i * 16, 16)] = zeros
    cs.wait()
    ct.wait()

    lane = lax.broadcasted_iota(jnp.int32, (16,), 0)
    lane64 = lane * _NB
    region = 16 * _NB
    ones = jnp.ones((16,), jnp.float32)
    unroll = 4

    def body(r, carry):
        vals = []
        for u in range(6):
            vals.append(s_v[r, pl.ds(u * 16, 16)])
            vals.append(t_v[r, pl.ds(u * 16, 16)])
        idxs = []
        for k, v in enumerate(vals):
            par = (k // 2) % 2
            tens = k % 2
            idxs.append(lane64 + (par * 2 * region + tens * region)
                        + v.astype(jnp.int32))
        for idx in idxs:
            plsc.addupdate_scatter(hist_v, [idx], ones)
        return carry

    lax.fori_loop(0, _RPW, body, 0)

    for t in range(2):
        for j in range(4):
            acc = hist_v[pl.ds(t * region + j * 16, 16)]
            for p in range(2):
                for l in range(16):
                    if p == 0 and l == 0:
                        continue
                    acc = acc + hist_v[pl.ds(p * 2 * region + t * region
                                             + l * _NB + j * 16, 16)]
            res_v[pl.ds(t * _NB + j * 16, 16)] = acc

    pltpu.sync_copy(res_v, out_hbm.at[pl.ds((q * 8 + row) * 128, 128)])


@functools.cache
def _hist_call():
    return pl.kernel(
        _sc_hist_kernel,
        out_type=jax.ShapeDtypeStruct((4 * 16 * _NB,), jnp.float32),
        mesh=plsc.VectorSubcoreMesh(core_axis_name="c", subcore_axis_name="s"),
        scratch_types=[
            pltpu.VMEM((_RPW, 96), jnp.float32),
            pltpu.VMEM((_RPW, 96), jnp.float32),
            pltpu.VMEM((4 * 16 * _NB,), jnp.float32),
            pltpu.VMEM((2 * _NB,), jnp.float32),
            pltpu.SemaphoreType.DMA,
        ],
        compiler_params=pltpu.CompilerParams(needs_layout_passes=False,
                                             use_tc_tiling_on_sc=True),
    )


def _compact_counts(h):
    ones8 = jnp.ones((1, _ROWS), jnp.float32)
    tot = jnp.dot(ones8, h, preferred_element_type=jnp.float32,
                  precision=lax.Precision.HIGHEST)
    pres = (tot > 0.5).astype(jnp.float32)
    iota_w = lax.broadcasted_iota(jnp.int32, (_NB, _NB), 0)
    iota_v = lax.broadcasted_iota(jnp.int32, (_NB, _NB), 1)
    le = (iota_w <= iota_v).astype(jnp.float32)
    cum = jnp.dot(pres, le, preferred_element_type=jnp.float32,
                  precision=lax.Precision.HIGHEST)
    iota_i = lax.broadcasted_iota(jnp.int32, (_NB, _NB), 0).astype(jnp.float32)
    q = pres * jnp.where(jnp.abs(cum - (iota_i + 1.0)) < 0.5, 1.0, 0.0)
    comp = lax.dot_general(h, q, (((1,), (1,)), ((), ())),
                           preferred_element_type=jnp.float32,
                           precision=lax.Precision.HIGHEST)
    comp = jnp.floor(comp + 0.5)
    nb = jnp.sum(pres)
    one_bin = nb == 1.0
    iota_row = lax.broadcasted_iota(jnp.int32, (1, _NB), 1).astype(jnp.float32)
    valid = jnp.where(one_bin,
                      jnp.where(iota_row < 2.0, 1.0, 0.0),
                      jnp.where(iota_row < nb, 1.0, 0.0))
    one = jnp.where(one_bin, jnp.ones((1, _NB), jnp.float32),
                    jnp.zeros((1, _NB), jnp.float32))
    return comp, valid, one


def _compact_kernel(p_ref, counts_ref, meta_ref):
    p2 = (p_ref[0:8] + p_ref[8:16] + p_ref[16:24] + p_ref[24:32])
    comp_s, valid_s, one_s = _compact_counts(p2[:, 0:_NB])
    comp_t, valid_t, one_t = _compact_counts(p2[:, _NB:2 * _NB])
    counts_ref[0:_ROWS] = comp_s
    counts_ref[_ROWS:2 * _ROWS] = comp_t
    meta_ref[0:1] = valid_s
    meta_ref[1:2] = valid_t
    meta_ref[2:3] = one_s
    meta_ref[3:4] = one_t


def kernel(feat_s, feat_t):
    fs = jnp.transpose(feat_s, (0, 2, 3, 1)).reshape(8 * 32 * 32, 96)
    ft = jnp.transpose(feat_t, (0, 2, 3, 1)).reshape(8 * 32 * 32, 96)
    partials = _hist_call()(fs, ft).reshape(32, 128)
    counts, meta = pl.pallas_call(
        _compact_kernel,
        out_shape=[
            jax.ShapeDtypeStruct((2 * _ROWS, _NB), jnp.float32),
            jax.ShapeDtypeStruct((4, _NB), jnp.float32),
        ],
    )(partials)
    comp_s = counts[0:_ROWS]
    comp_t = counts[_ROWS:2 * _ROWS]
    valid_s = meta[0] > 0.5
    valid_t = meta[1] > 0.5
    one_s = meta[2, 0] > 0.5
    one_t = meta[3, 0] > 0.5
    logit_s = jnp.where(one_s, jnp.zeros_like(comp_s),
                        jnp.log(comp_s + 1e-8))
    logit_t = jnp.where(one_t, jnp.zeros_like(comp_t),
                        jnp.log(comp_t + 1e-8))
    p_s = jax.nn.log_softmax(jnp.where(valid_s, logit_s / _T, -jnp.inf),
                             axis=1)
    p_t = jax.nn.softmax(jnp.where(valid_t, logit_t / _T, -jnp.inf), axis=1)
    valid = valid_s & valid_t
    kl_terms = jnp.where(valid, p_t * (jnp.log(p_t) - p_s), 0.0)
    return jnp.sum(kl_terms) * (_T ** 2) / _ROWS


# speedup vs baseline: 1.8129x; 1.0839x over previous
import functools

import jax
import jax.numpy as jnp
from jax import lax
from jax.experimental import pallas as pl
from jax.experimental.pallas import tpu as pltpu
from jax.experimental.pallas import tpu_sc as plsc

_T = 4.0
_NB = 64
_ROWS = 8
_NPR = 96 * 32 * 32
_SPLIT = 4
_CHUNK = _NPR // _SPLIT
_ITERS = _CHUNK // 16
_NC = 2
_NS = 16


_RPW = 256


def _sc_hist_kernel(fs_hbm, ft_hbm, out_hbm, s_v, t_v, hist_v, res_v,
                    sem0, sem1):
    wid = lax.axis_index("s") * _NC + lax.axis_index("c")
    row = wid // _SPLIT
    q = wid % _SPLIT
    r0 = wid * _RPW
    half = _RPW // 2
    cs0 = pltpu.async_copy(fs_hbm.at[pl.ds(r0, half), :],
                           s_v.at[pl.ds(0, half), :], sem0)
    ct0 = pltpu.async_copy(ft_hbm.at[pl.ds(r0, half), :],
                           t_v.at[pl.ds(0, half), :], sem0)
    cs1 = pltpu.async_copy(fs_hbm.at[pl.ds(r0 + half, half), :],
                           s_v.at[pl.ds(half, half), :], sem1)
    ct1 = pltpu.async_copy(ft_hbm.at[pl.ds(r0 + half, half), :],
                           t_v.at[pl.ds(half, half), :], sem1)

    zeros = jnp.zeros((16,), jnp.float32)
    for i in range((4 * 16 * _NB) // 16):
        hist_v[pl.ds(i * 16, 16)] = zeros
    cs0.wait()
    ct0.wait()

    lane = lax.broadcasted_iota(jnp.int32, (16,), 0)
    lane64 = lane * _NB
    region = 16 * _NB
    ones = jnp.ones((16,), jnp.float32)
    unroll = 4

    def body(r, carry):
        vals = []
        for u in range(6):
            vals.append(s_v[r, pl.ds(u * 16, 16)])
            vals.append(t_v[r, pl.ds(u * 16, 16)])
        idxs = []
        for k, v in enumerate(vals):
            par = (k // 2) % 2
            tens = k % 2
            idxs.append(lane64 + (par * 2 * region + tens * region)
                        + v.astype(jnp.int32))
        for idx in idxs:
            plsc.addupdate_scatter(hist_v, [idx], ones)
        return carry

    lax.fori_loop(0, _RPW // 2, body, 0)
    cs1.wait()
    ct1.wait()
    lax.fori_loop(_RPW // 2, _RPW, body, 0)

    for t in range(2):
        for j in range(4):
            acc = hist_v[pl.ds(t * region + j * 16, 16)]
            for p in range(2):
                for l in range(16):
                    if p == 0 and l == 0:
                        continue
                    acc = acc + hist_v[pl.ds(p * 2 * region + t * region
                                             + l * _NB + j * 16, 16)]
            res_v[pl.ds(t * _NB + j * 16, 16)] = acc

    pltpu.sync_copy(res_v, out_hbm.at[pl.ds((q * 8 + row) * 128, 128)])


@functools.cache
def _hist_call():
    return pl.kernel(
        _sc_hist_kernel,
        out_type=jax.ShapeDtypeStruct((4 * 16 * _NB,), jnp.float32),
        mesh=plsc.VectorSubcoreMesh(core_axis_name="c", subcore_axis_name="s"),
        scratch_types=[
            pltpu.VMEM((_RPW, 96), jnp.float32),
            pltpu.VMEM((_RPW, 96), jnp.float32),
            pltpu.VMEM((4 * 16 * _NB,), jnp.float32),
            pltpu.VMEM((2 * _NB,), jnp.float32),
            pltpu.SemaphoreType.DMA,
            pltpu.SemaphoreType.DMA,
        ],
        compiler_params=pltpu.CompilerParams(needs_layout_passes=False,
                                             use_tc_tiling_on_sc=True),
    )


def _compact_counts(h):
    ones8 = jnp.ones((1, _ROWS), jnp.float32)
    tot = jnp.dot(ones8, h, preferred_element_type=jnp.float32,
                  precision=lax.Precision.HIGHEST)
    pres = (tot > 0.5).astype(jnp.float32)
    iota_w = lax.broadcasted_iota(jnp.int32, (_NB, _NB), 0)
    iota_v = lax.broadcasted_iota(jnp.int32, (_NB, _NB), 1)
    le = (iota_w <= iota_v).astype(jnp.float32)
    cum = jnp.dot(pres, le, preferred_element_type=jnp.float32,
                  precision=lax.Precision.HIGHEST)
    iota_i = lax.broadcasted_iota(jnp.int32, (_NB, _NB), 0).astype(jnp.float32)
    q = pres * jnp.where(jnp.abs(cum - (iota_i + 1.0)) < 0.5, 1.0, 0.0)
    comp = lax.dot_general(h, q, (((1,), (1,)), ((), ())),
                           preferred_element_type=jnp.float32,
                           precision=lax.Precision.HIGHEST)
    comp = jnp.floor(comp + 0.5)
    nb = jnp.sum(pres)
    one_bin = nb == 1.0
    iota_row = lax.broadcasted_iota(jnp.int32, (1, _NB), 1).astype(jnp.float32)
    valid = jnp.where(one_bin,
                      jnp.where(iota_row < 2.0, 1.0, 0.0),
                      jnp.where(iota_row < nb, 1.0, 0.0))
    one = jnp.where(one_bin, jnp.ones((1, _NB), jnp.float32),
                    jnp.zeros((1, _NB), jnp.float32))
    return comp, valid, one


def _compact_kernel(p_ref, x_ref, meta_ref):
    p2 = (p_ref[0:8] + p_ref[8:16] + p_ref[16:24] + p_ref[24:32])
    comp_s, valid_s, one_s = _compact_counts(p2[:, 0:_NB])
    comp_t, valid_t, one_t = _compact_counts(p2[:, _NB:2 * _NB])
    logit_s = jnp.where(one_s > 0.0, 0.0, jnp.log(comp_s + 1e-8))
    logit_t = jnp.where(one_t > 0.0, 0.0, jnp.log(comp_t + 1e-8))
    x_ref[0:_ROWS] = jnp.where(valid_s > 0.0, logit_s / _T, -jnp.inf)
    x_ref[_ROWS:2 * _ROWS] = jnp.where(valid_t > 0.0, logit_t / _T, -jnp.inf)
    meta_ref[0:1] = valid_s
    meta_ref[1:2] = valid_t


def kernel(feat_s, feat_t):
    fs = jnp.transpose(feat_s, (0, 2, 3, 1)).reshape(8 * 32 * 32, 96)
    ft = jnp.transpose(feat_t, (0, 2, 3, 1)).reshape(8 * 32 * 32, 96)
    partials = _hist_call()(fs, ft).reshape(32, 128)
    x, meta = pl.pallas_call(
        _compact_kernel,
        out_shape=[
            jax.ShapeDtypeStruct((2 * _ROWS, _NB), jnp.float32),
            jax.ShapeDtypeStruct((2, _NB), jnp.float32),
        ],
    )(partials)
    p_s = jax.nn.log_softmax(x[0:_ROWS], axis=1)
    p_t = jax.nn.softmax(x[_ROWS:2 * _ROWS], axis=1)
    valid = (meta[0] > 0.5) & (meta[1] > 0.5)
    kl_terms = jnp.where(valid, p_t * (jnp.log(p_t) - p_s), 0.0)
    return jnp.sum(kl_terms) * (_T ** 2) / _ROWS
